# baseline (speedup 1.0000x reference)
import jax
import jax.numpy as jnp
from jax.experimental import pallas as pl
from jax.experimental.pallas import tpu as pltpu

_NUM_CLASSES = 80
_NUM_ANCHORS = 3
_STRIDE = 8.0
_NCH = _NUM_CLASSES + 7
_S = 4


def _yolo_body(anch_ref, x_ref, o_ref):
    a = pl.program_id(1)
    s = pl.program_id(2)
    aw = anch_ref[a, 0]
    ah = anch_ref[a, 1]
    v = x_ref[0]
    c = jax.lax.broadcasted_iota(jnp.int32, v.shape, 0)
    gx = jax.lax.broadcasted_iota(jnp.int32, v.shape, 2).astype(jnp.float32)
    gy = jax.lax.broadcasted_iota(jnp.int32, v.shape, 1).astype(jnp.float32)
    gy = gy + (v.shape[1] * s).astype(jnp.float32)
    sgn = jnp.where((c == 2) | (c == 3), 1.0, -1.0)
    ca = jnp.where(c < 2, _STRIDE, jnp.where(c >= 6, 1.0, 0.0))
    cb = jnp.where(c == 2, aw, jnp.where(c == 3, ah, 0.0))
    cc = jnp.where((c == 4) | (c == 5), 1.0, 0.0)
    add = _STRIDE * jnp.where(c == 0, gx, jnp.where(c == 1, gy, 0.0))
    e = jnp.exp(v * sgn)
    sig = 1.0 / (1.0 + e)
    w = ca * sig + cb * e + cc * v + add
    o_ref[0, 0] = jnp.transpose(w, (1, 2, 0))


def kernel(x, anchors):
    B, C, G, _ = x.shape
    nA, nCh = _NUM_ANCHORS, _NCH
    R = G // _S

    out = pl.pallas_call(
        _yolo_body,
        grid=(B, nA, _S),
        in_specs=[
            pl.BlockSpec(memory_space=pltpu.SMEM),
            pl.BlockSpec((1, nCh, R, G), lambda b, a, s: (b, a, s, 0)),
        ],
        out_specs=pl.BlockSpec(
            (1, 1, R, G, nCh), lambda b, a, s: (b, a, s, 0, 0)),
        out_shape=jax.ShapeDtypeStruct((B, nA, G, G, nCh), jnp.float32),
        compiler_params=pltpu.CompilerParams(
            dimension_semantics=("parallel", "parallel", "parallel"),
        ),
    )(anchors, x)
    return out.reshape(B, nA * G * G, nCh)


# speedup vs baseline: 1.6037x; 1.6037x over previous
import jax
import jax.numpy as jnp
from jax.experimental import pallas as pl
from jax.experimental.pallas import tpu as pltpu

_NUM_CLASSES = 80
_NUM_ANCHORS = 3
_STRIDE = 8.0
_NCH = _NUM_CLASSES + 7


def _yolo_body(anch_ref, x_ref, o_ref):
    for bi in range(2):
      for i in range(_NUM_ANCHORS):
        aw = anch_ref[i, 0]
        ah = anch_ref[i, 1]
        v = x_ref[bi, i * _NCH:(i + 1) * _NCH]
        c = jax.lax.broadcasted_iota(jnp.int32, v.shape, 0)
        gx = jax.lax.broadcasted_iota(jnp.int32, v.shape, 2).astype(jnp.float32)
        gy = jax.lax.broadcasted_iota(jnp.int32, v.shape, 1).astype(jnp.float32)
        sgn = jnp.where((c == 2) | (c == 3), 1.0, -1.0)
        ca = jnp.where(c < 2, _STRIDE, jnp.where(c >= 6, 1.0, 0.0))
        cb = jnp.where(c == 2, aw, jnp.where(c == 3, ah, 0.0))
        cc = jnp.where((c == 4) | (c == 5), 1.0, 0.0)
        add = _STRIDE * jnp.where(c == 0, gx, jnp.where(c == 1, gy, 0.0))
        e = jnp.exp(v * sgn)
        sig = 1.0 / (1.0 + e)
        w = ca * sig + cb * e + cc * v + add
        o_ref[bi, i] = jnp.transpose(w, (1, 2, 0))


def kernel(x, anchors):
    B, C, G, _ = x.shape
    nA, nCh = _NUM_ANCHORS, _NCH

    out = pl.pallas_call(
        _yolo_body,
        grid=(B // 2,),
        in_specs=[
            pl.BlockSpec(memory_space=pltpu.SMEM),
            pl.BlockSpec((2, C, G, G), lambda b: (b, 0, 0, 0)),
        ],
        out_specs=pl.BlockSpec((2, nA, G, G, nCh), lambda b: (b, 0, 0, 0, 0)),
        out_shape=jax.ShapeDtypeStruct((B, nA, G, G, nCh), jnp.float32),
        compiler_params=pltpu.CompilerParams(
            dimension_semantics=("arbitrary",),
            vmem_limit_bytes=100 * 1024 * 1024,
        ),
    )(anchors, x)
    return out.reshape(B, nA * G * G, nCh)
